# baseline (speedup 1.0000x reference)
import functools

import jax
import jax.numpy as jnp
from jax import lax
from jax.experimental import pallas as pl
from jax.experimental.pallas import tpu as pltpu
from jax.experimental.pallas import tpu_sc as plsc

NUM_ROWS = 1000000
D = 64
B = 16384
W = 128

NC = 2
NS = 16
L = 16
NW = NC * NS
BPW = B // NW
NSLOT = 12
DELTA = NSLOT - 1
SB = 16


def _gather_body(srt_hbm, table, emb, stask, vsrt, wbuf, stage, wsems, bsem):
    wid = lax.axis_index("s") * NC + lax.axis_index("c")
    base = wid * BPW
    lane = lax.iota(jnp.int32, L)

    pltpu.sync_copy(srt_hbm.at[pl.ds(base, BPW)], vsrt)

    def spill(g, carry):
        v = vsrt[pl.ds(g * L, L)]
        for i in range(L):
            stask[g * L + i] = v[i]
        return carry

    lax.fori_loop(0, BPW // L, spill, 0)

    def fire(f, q):
        t = lax.rem(f, NSLOT)
        col = pl.multiple_of(q << 7, W)
        pltpu.async_copy(table.at[pl.ds(0, D), pl.ds(col, W)],
                         wbuf.at[t], wsems.at[t])

    def drain(t):
        pltpu.make_async_copy(table.at[pl.ds(0, D), pl.ds(0, W)],
                              wbuf.at[t], wsems.at[t]).wait()

    def prol(m, f):
        qm = stask[m] >> 7
        qp = stask[jnp.maximum(m - 1, 0)] >> 7
        head = jnp.logical_or(m == 0, qm != qp)

        @pl.when(head)
        def _():
            fire(f, qm)

        return f + head.astype(jnp.int32)

    f0 = lax.fori_loop(0, DELTA, prol, 0)

    def task(n, carry):
        s, f = carry
        rec = stask[n]
        qn = rec >> 7
        lv = rec & 127
        qp = stask[jnp.maximum(n - 1, 0)] >> 7
        head = jnp.logical_or(n == 0, qn != qp)
        s = s + head.astype(jnp.int32)

        @pl.when(head)
        def _():
            drain(lax.rem(s, NSLOT))

        m = jnp.minimum(n + DELTA, BPW - 1)
        qm = stask[m] >> 7
        qmp = stask[m - 1] >> 7
        headm = jnp.logical_and(n + DELTA < BPW, qm != qmp)

        @pl.when(headm)
        def _():
            fire(f, qm)

        f = f + headm.astype(jnp.int32)

        @pl.when(jnp.logical_and((n & (SB - 1)) == 0, n >= 2 * SB))
        def _():
            pltpu.make_async_copy(emb.at[pl.ds(0, SB)], stage.at[0],
                                  bsem).wait()

        t = lax.rem(s, NSLOT)
        bslot = lax.rem(n >> 4, 2)
        k = n & (SB - 1)
        lvec = jnp.broadcast_to(lv, (L,))
        for c in range(D // L):
            u = plsc.load_gather(wbuf.at[t], [lane + (c * L), lvec])
            stage[bslot, k, pl.ds(c * L, L)] = u

        @pl.when((n & (SB - 1)) == SB - 1)
        def _():
            off = pl.multiple_of(base + n - (SB - 1), SB)
            pltpu.async_copy(stage.at[bslot], emb.at[pl.ds(off, SB)], bsem)

        return (s, f)

    lax.fori_loop(0, BPW, task, (jnp.int32(-1), f0))
    for i in range(2):
        pltpu.make_async_copy(emb.at[pl.ds(0, SB)], stage.at[0], bsem).wait()


CB = 128
NCHUNK = BPW // CB


def _dot_body(invu_hbm, invi_hbm, embu_hbm, embi_hbm, out_hbm,
              idxu, idxi, bu, bi, out_v, semu, semi):
    wid = lax.axis_index("s") * NC + lax.axis_index("c")
    base = wid * BPW
    lane = lax.iota(jnp.int32, L)

    for j in range(NCHUNK):
        pltpu.sync_copy(invu_hbm.at[pl.ds(base + j * CB, CB)], idxu.at[j])
        pltpu.sync_copy(invi_hbm.at[pl.ds(base + j * CB, CB)], idxi.at[j])

    dnums = lax.GatherDimensionNumbers(
        offset_dims=(), collapsed_slice_dims=(0,), start_index_map=(0,))

    def permute(v, idx):
        return lax.gather(v, idx[:, None], dimension_numbers=dnums,
                          slice_sizes=(1,),
                          mode=lax.GatherScatterMode.PROMISE_IN_BOUNDS)

    def sum_lanes(v):
        for sh in (8, 4, 2, 1):
            v = v + permute(v, lane ^ sh)
        return v

    def fire(j):
        p = j & 1
        pltpu.async_copy(embu_hbm.at[idxu.at[j]], bu.at[p], semu.at[p])
        pltpu.async_copy(embi_hbm.at[idxi.at[j]], bi.at[p], semi.at[p])

    fire(0)
    for j in range(NCHUNK):
        if j + 1 < NCHUNK:
            fire(j + 1)
        p = j & 1
        pltpu.make_async_copy(embu_hbm.at[pl.ds(0, CB)], bu.at[p],
                              semu.at[p]).wait()
        pltpu.make_async_copy(embi_hbm.at[pl.ds(0, CB)], bi.at[p],
                              semi.at[p]).wait()

        def group(g, carry, p=p, j=j):
            vec = jnp.zeros((L,), jnp.float32)
            for i in range(L):
                k = g * L + i
                acc = bu[p, k, pl.ds(0, L)] * bi[p, k, pl.ds(0, L)]
                for c in range(1, D // L):
                    acc = acc + (bu[p, k, pl.ds(c * L, L)]
                                 * bi[p, k, pl.ds(c * L, L)])
                vec = jnp.where(lane == i, sum_lanes(acc), vec)
            out_v[pl.ds(j * CB + g * L, L)] = vec
            return carry

        lax.fori_loop(0, CB // L, group, 0)

    for t in range(BPW // L):
        x = out_v[pl.ds(t * L, L)]
        out_v[pl.ds(t * L, L)] = 1.0 / (1.0 + jnp.exp(-x))
    pltpu.sync_copy(out_v, out_hbm.at[pl.ds(base, BPW)])


@jax.jit
def kernel(users, items, user_table, item_table):
    users = users.astype(jnp.int32)
    items = items.astype(jnp.int32)
    mesh = plsc.VectorSubcoreMesh(core_axis_name="c", subcore_axis_name="s")

    gather = pl.kernel(
        _gather_body,
        out_type=jax.ShapeDtypeStruct((B, W), jnp.float32),
        mesh=mesh,
        compiler_params=pltpu.CompilerParams(needs_layout_passes=False),
        scratch_types=[
            pltpu.SMEM((BPW,), jnp.int32),
            pltpu.VMEM((BPW,), jnp.int32),
            pltpu.VMEM((NSLOT, D, W), jnp.float32),
            pltpu.VMEM((2, SB, W), jnp.float32),
            pltpu.SemaphoreType.DMA((NSLOT,)),
            pltpu.SemaphoreType.DMA,
        ],
    )

    iot = jnp.arange(B, dtype=jnp.int32)

    ord_u = jnp.argsort(users).astype(jnp.int32)
    srt_u = users[ord_u]
    inv_u = jnp.zeros((B,), jnp.int32).at[ord_u].set(
        iot, unique_indices=True, mode="promise_in_bounds")
    embu = gather(srt_u, user_table.T)

    ord_i = jnp.argsort(items).astype(jnp.int32)
    srt_i = items[ord_i]
    inv_i = jnp.zeros((B,), jnp.int32).at[ord_i].set(
        iot, unique_indices=True, mode="promise_in_bounds")
    embi = gather(srt_i, item_table.T)

    dot = pl.kernel(
        _dot_body,
        out_type=jax.ShapeDtypeStruct((B,), jnp.float32),
        mesh=mesh,
        compiler_params=pltpu.CompilerParams(needs_layout_passes=False),
        scratch_types=[
            pltpu.VMEM((NCHUNK, CB), jnp.int32),
            pltpu.VMEM((NCHUNK, CB), jnp.int32),
            pltpu.VMEM((2, CB, W), jnp.float32),
            pltpu.VMEM((2, CB, W), jnp.float32),
            pltpu.VMEM((BPW,), jnp.float32),
            pltpu.SemaphoreType.DMA((2,)),
            pltpu.SemaphoreType.DMA((2,)),
        ],
    )
    return dot(inv_u, inv_i, embu, embi)


# speedup vs baseline: 1.0069x; 1.0069x over previous
import functools

import jax
import jax.numpy as jnp
from jax import lax
from jax.experimental import pallas as pl
from jax.experimental.pallas import tpu as pltpu
from jax.experimental.pallas import tpu_sc as plsc

NUM_ROWS = 1000000
D = 64
B = 16384
W = 128

NC = 2
NS = 16
L = 16
NW = NC * NS
BPW = B // NW
NSLOT = 12
DELTA = NSLOT - 1
SB = 16


def _gather_body(srt_hbm, table, emb, stask, vsrt, wbuf, stage, wsems, bsem):
    wid = lax.axis_index("s") * NC + lax.axis_index("c")
    base = wid * BPW
    lane = lax.iota(jnp.int32, L)

    pltpu.sync_copy(srt_hbm.at[pl.ds(base, BPW)], vsrt)

    def spill(g, carry):
        v = vsrt[pl.ds(g * L, L)]
        for i in range(L):
            stask[g * L + i] = v[i]
        return carry

    lax.fori_loop(0, BPW // L, spill, 0)

    def fire(f, q):
        t = lax.rem(f, NSLOT)
        col = pl.multiple_of(q << 7, W)
        pltpu.async_copy(table.at[pl.ds(0, D), pl.ds(col, W)],
                         wbuf.at[t], wsems.at[t])

    def drain(t):
        pltpu.make_async_copy(table.at[pl.ds(0, D), pl.ds(0, W)],
                              wbuf.at[t], wsems.at[t]).wait()

    def prol(m, f):
        qm = stask[m] >> 7
        qp = stask[jnp.maximum(m - 1, 0)] >> 7
        head = jnp.logical_or(m == 0, qm != qp)

        @pl.when(head)
        def _():
            fire(f, qm)

        return f + head.astype(jnp.int32)

    f0 = lax.fori_loop(0, DELTA, prol, 0)

    def task(n, carry):
        s, f = carry
        rec = stask[n]
        qn = rec >> 7
        lv = rec & 127
        qp = stask[jnp.maximum(n - 1, 0)] >> 7
        head = jnp.logical_or(n == 0, qn != qp)
        s = s + head.astype(jnp.int32)

        @pl.when(head)
        def _():
            drain(lax.rem(s, NSLOT))

        m = jnp.minimum(n + DELTA, BPW - 1)
        qm = stask[m] >> 7
        qmp = stask[m - 1] >> 7
        headm = jnp.logical_and(n + DELTA < BPW, qm != qmp)

        @pl.when(headm)
        def _():
            fire(f, qm)

        f = f + headm.astype(jnp.int32)

        @pl.when(jnp.logical_and((n & (SB - 1)) == 0, n >= 2 * SB))
        def _():
            pltpu.make_async_copy(emb.at[pl.ds(0, SB)], stage.at[0],
                                  bsem).wait()

        t = lax.rem(s, NSLOT)
        bslot = lax.rem(n >> 4, 2)
        k = n & (SB - 1)
        lvec = jnp.broadcast_to(lv, (L,))
        for c in range(D // L):
            u = plsc.load_gather(wbuf.at[t], [lane + (c * L), lvec])
            stage[bslot, k, pl.ds(c * L, L)] = u

        @pl.when((n & (SB - 1)) == SB - 1)
        def _():
            off = pl.multiple_of(base + n - (SB - 1), SB)
            pltpu.async_copy(stage.at[bslot], emb.at[pl.ds(off, SB)], bsem)

        return (s, f)

    lax.fori_loop(0, BPW, task, (jnp.int32(-1), f0))
    for i in range(2):
        pltpu.make_async_copy(emb.at[pl.ds(0, SB)], stage.at[0], bsem).wait()


CB = 128
NCHUNK = BPW // CB


def _dot_body(invu_hbm, invi_hbm, embu_hbm, embi_hbm, out_hbm,
              idxu, idxi, bu, bi, out_v, semu, semi):
    wid = lax.axis_index("s") * NC + lax.axis_index("c")
    base = wid * BPW
    lane = lax.iota(jnp.int32, L)

    for j in range(NCHUNK):
        pltpu.sync_copy(invu_hbm.at[pl.ds(base + j * CB, CB)], idxu.at[j])
        pltpu.sync_copy(invi_hbm.at[pl.ds(base + j * CB, CB)], idxi.at[j])

    dnums = lax.GatherDimensionNumbers(
        offset_dims=(), collapsed_slice_dims=(0,), start_index_map=(0,))

    def permute(v, idx):
        return lax.gather(v, idx[:, None], dimension_numbers=dnums,
                          slice_sizes=(1,),
                          mode=lax.GatherScatterMode.PROMISE_IN_BOUNDS)

    def sum_lanes(v):
        for sh in (8, 4, 2, 1):
            v = v + permute(v, lane ^ sh)
        return v

    def fire(j):
        p = j & 1
        pltpu.async_copy(embu_hbm.at[idxu.at[j]], bu.at[p], semu.at[p])
        pltpu.async_copy(embi_hbm.at[idxi.at[j]], bi.at[p], semi.at[p])

    fire(0)
    for j in range(NCHUNK):
        if j + 1 < NCHUNK:
            fire(j + 1)
        p = j & 1
        pltpu.make_async_copy(embu_hbm.at[pl.ds(0, CB)], bu.at[p],
                              semu.at[p]).wait()
        pltpu.make_async_copy(embi_hbm.at[pl.ds(0, CB)], bi.at[p],
                              semi.at[p]).wait()

        def group(g, carry, p=p, j=j):
            vec = jnp.zeros((L,), jnp.float32)
            for i in range(L):
                k = g * L + i
                acc = bu[p, k, pl.ds(0, L)] * bi[p, k, pl.ds(0, L)]
                for c in range(1, D // L):
                    acc = acc + (bu[p, k, pl.ds(c * L, L)]
                                 * bi[p, k, pl.ds(c * L, L)])
                vec = jnp.where(lane == i, sum_lanes(acc), vec)
            out_v[pl.ds(j * CB + g * L, L)] = vec
            return carry

        lax.fori_loop(0, CB // L, group, 0)

    for t in range(BPW // L):
        x = out_v[pl.ds(t * L, L)]
        out_v[pl.ds(t * L, L)] = 1.0 / (1.0 + jnp.exp(-x))
    pltpu.sync_copy(out_v, out_hbm.at[pl.ds(base, BPW)])


@jax.jit
def kernel(users, items, user_table, item_table):
    users = users.astype(jnp.int32)
    items = items.astype(jnp.int32)
    mesh = plsc.VectorSubcoreMesh(core_axis_name="c", subcore_axis_name="s")

    gather = pl.kernel(
        _gather_body,
        out_type=jax.ShapeDtypeStruct((B, W), jnp.float32),
        mesh=mesh,
        compiler_params=pltpu.CompilerParams(needs_layout_passes=False),
        scratch_types=[
            pltpu.SMEM((BPW,), jnp.int32),
            pltpu.VMEM((BPW,), jnp.int32),
            pltpu.VMEM((NSLOT, D, W), jnp.float32),
            pltpu.VMEM((2, SB, W), jnp.float32),
            pltpu.SemaphoreType.DMA((NSLOT,)),
            pltpu.SemaphoreType.DMA,
        ],
    )

    ord_u = jnp.argsort(users).astype(jnp.int32)
    srt_u = users[ord_u]
    inv_u = jnp.argsort(ord_u).astype(jnp.int32)
    embu = gather(srt_u, user_table.T)

    ord_i = jnp.argsort(items).astype(jnp.int32)
    srt_i = items[ord_i]
    inv_i = jnp.argsort(ord_i).astype(jnp.int32)
    embi = gather(srt_i, item_table.T)

    dot = pl.kernel(
        _dot_body,
        out_type=jax.ShapeDtypeStruct((B,), jnp.float32),
        mesh=mesh,
        compiler_params=pltpu.CompilerParams(needs_layout_passes=False),
        scratch_types=[
            pltpu.VMEM((NCHUNK, CB), jnp.int32),
            pltpu.VMEM((NCHUNK, CB), jnp.int32),
            pltpu.VMEM((2, CB, W), jnp.float32),
            pltpu.VMEM((2, CB, W), jnp.float32),
            pltpu.VMEM((BPW,), jnp.float32),
            pltpu.SemaphoreType.DMA((2,)),
            pltpu.SemaphoreType.DMA((2,)),
        ],
    )
    return dot(inv_u, inv_i, embu, embi)
